# baseline (speedup 1.0000x reference)
import jax
import jax.numpy as jnp
from jax.experimental import pallas as pl
from jax.experimental.pallas import tpu as pltpu

_E, _CAP, _D = 64, 1024, 512
_H = 4 * _D
_CB = 128


def _ffn_body(vl_ref, x_ref, w1_ref, b1_ref, w2_ref, b2_ref, o_ref):
    e = pl.program_id(0)
    c = pl.program_id(1)
    valid = vl_ref[e]
    base = c * _CB

    @pl.when(base < valid)
    def _compute():
        x = x_ref[0]
        h = jnp.dot(x, w1_ref[0], preferred_element_type=jnp.float32)
        h = h + b1_ref[0]
        h = 0.5 * h * (1.0 + jax.lax.erf(h * 0.7071067811865476))
        y = jnp.dot(h, w2_ref[0], preferred_element_type=jnp.float32)
        y = y + b2_ref[0]
        rows = base + jax.lax.broadcasted_iota(jnp.int32, (_CB, 1), 0)
        o_ref[0] = jnp.where(rows < valid, y, 0.0)

    @pl.when(base >= valid)
    def _zero():
        o_ref[...] = jnp.zeros_like(o_ref)


def kernel(packed_inputs, valid_load, w1, b1, w2, b2):
    vl = valid_load.astype(jnp.int32)
    b1r = b1.reshape(_E, 1, _H)
    b2r = b2.reshape(_E, 1, _D)

    grid = (_E, _CAP // _CB)

    def _xmap(e, c, vl_ref):
        last_valid = jnp.maximum((vl_ref[e] + _CB - 1) // _CB - 1, 0)
        return (e, jnp.minimum(c, last_valid), 0)

    def _emap(e, c, vl_ref):
        return (e, 0, 0)

    out = pl.pallas_call(
        _ffn_body,
        out_shape=jax.ShapeDtypeStruct((_E, _CAP, _D), jnp.float32),
        grid_spec=pltpu.PrefetchScalarGridSpec(
            num_scalar_prefetch=1,
            grid=grid,
            in_specs=[
                pl.BlockSpec((1, _CB, _D), _xmap),
                pl.BlockSpec((1, _D, _H), _emap),
                pl.BlockSpec((1, 1, _H), _emap),
                pl.BlockSpec((1, _H, _D), _emap),
                pl.BlockSpec((1, 1, _D), _emap),
            ],
            out_specs=pl.BlockSpec((1, _CB, _D), lambda e, c, vl_ref: (e, c, 0)),
        ),
        compiler_params=pltpu.CompilerParams(
            dimension_semantics=("parallel", "arbitrary"),
            vmem_limit_bytes=56 * 1024 * 1024,
        ),
        name="fused_group_ffn",
    )(vl, packed_inputs, w1, b1r, w2, b2r)
    return out


# speedup vs baseline: 1.2631x; 1.2631x over previous
import jax
import jax.numpy as jnp
from jax.experimental import pallas as pl
from jax.experimental.pallas import tpu as pltpu

_E, _CAP, _D = 64, 1024, 512
_H = 4 * _D
_CB = 512


def _ffn_body(vl_ref, x_ref, w1_ref, b1_ref, w2_ref, b2_ref, o_ref):
    e = pl.program_id(0)
    c = pl.program_id(1)
    valid = vl_ref[e]
    base = c * _CB

    @pl.when(base < valid)
    def _compute():
        x = x_ref[0]
        h = jnp.dot(x, w1_ref[0], preferred_element_type=jnp.float32)
        h = h + b1_ref[0]
        h = 0.5 * h * (1.0 + jax.lax.erf(h * 0.7071067811865476))
        y = jnp.dot(h, w2_ref[0], preferred_element_type=jnp.float32)
        y = y + b2_ref[0]
        rows = base + jax.lax.broadcasted_iota(jnp.int32, (_CB, 1), 0)
        o_ref[0] = jnp.where(rows < valid, y, 0.0)

    @pl.when(base >= valid)
    def _zero():
        o_ref[...] = jnp.zeros_like(o_ref)


def kernel(packed_inputs, valid_load, w1, b1, w2, b2):
    vl = valid_load.astype(jnp.int32)
    b1r = b1.reshape(_E, 1, _H)
    b2r = b2.reshape(_E, 1, _D)

    grid = (_E, _CAP // _CB)

    def _xmap(e, c, vl_ref):
        last_valid = jnp.maximum((vl_ref[e] + _CB - 1) // _CB - 1, 0)
        return (e, jnp.minimum(c, last_valid), 0)

    def _emap(e, c, vl_ref):
        return (e, 0, 0)

    out = pl.pallas_call(
        _ffn_body,
        out_shape=jax.ShapeDtypeStruct((_E, _CAP, _D), jnp.float32),
        grid_spec=pltpu.PrefetchScalarGridSpec(
            num_scalar_prefetch=1,
            grid=grid,
            in_specs=[
                pl.BlockSpec((1, _CB, _D), _xmap),
                pl.BlockSpec((1, _D, _H), _emap),
                pl.BlockSpec((1, 1, _H), _emap),
                pl.BlockSpec((1, _H, _D), _emap),
                pl.BlockSpec((1, 1, _D), _emap),
            ],
            out_specs=pl.BlockSpec((1, _CB, _D), lambda e, c, vl_ref: (e, c, 0)),
        ),
        compiler_params=pltpu.CompilerParams(
            dimension_semantics=("parallel", "arbitrary"),
            vmem_limit_bytes=56 * 1024 * 1024,
        ),
        name="fused_group_ffn",
    )(vl, packed_inputs, w1, b1r, w2, b2r)
    return out


# speedup vs baseline: 1.8067x; 1.4304x over previous
import jax
import jax.numpy as jnp
from jax.experimental import pallas as pl
from jax.experimental.pallas import tpu as pltpu

_E, _CAP, _D = 64, 1024, 512
_H = 4 * _D
_CB = 1024


def _ffn_body(vl_ref, x_ref, w1_ref, b1_ref, w2_ref, b2_ref, o_ref):
    e = pl.program_id(0)
    c = pl.program_id(1)
    valid = vl_ref[e]
    base = c * _CB

    @pl.when(base < valid)
    def _compute():
        x = x_ref[0]
        h = jnp.dot(x, w1_ref[0], preferred_element_type=jnp.float32)
        h = h + b1_ref[0]
        h = 0.5 * h * (1.0 + jax.lax.erf(h * 0.7071067811865476))
        y = jnp.dot(h, w2_ref[0], preferred_element_type=jnp.float32)
        y = y + b2_ref[0]
        rows = base + jax.lax.broadcasted_iota(jnp.int32, (_CB, 1), 0)
        o_ref[0] = jnp.where(rows < valid, y, 0.0)

    @pl.when(base >= valid)
    def _zero():
        o_ref[...] = jnp.zeros_like(o_ref)


def kernel(packed_inputs, valid_load, w1, b1, w2, b2):
    vl = valid_load.astype(jnp.int32)
    b1r = b1.reshape(_E, 1, _H)
    b2r = b2.reshape(_E, 1, _D)

    grid = (_E, _CAP // _CB)

    def _xmap(e, c, vl_ref):
        last_valid = jnp.maximum((vl_ref[e] + _CB - 1) // _CB - 1, 0)
        return (e, jnp.minimum(c, last_valid), 0)

    def _emap(e, c, vl_ref):
        return (e, 0, 0)

    out = pl.pallas_call(
        _ffn_body,
        out_shape=jax.ShapeDtypeStruct((_E, _CAP, _D), jnp.float32),
        grid_spec=pltpu.PrefetchScalarGridSpec(
            num_scalar_prefetch=1,
            grid=grid,
            in_specs=[
                pl.BlockSpec((1, _CB, _D), _xmap),
                pl.BlockSpec((1, _D, _H), _emap),
                pl.BlockSpec((1, 1, _H), _emap),
                pl.BlockSpec((1, _H, _D), _emap),
                pl.BlockSpec((1, 1, _D), _emap),
            ],
            out_specs=pl.BlockSpec((1, _CB, _D), lambda e, c, vl_ref: (e, c, 0)),
        ),
        compiler_params=pltpu.CompilerParams(
            dimension_semantics=("parallel", "arbitrary"),
            vmem_limit_bytes=56 * 1024 * 1024,
        ),
        name="fused_group_ffn",
    )(vl, packed_inputs, w1, b1r, w2, b2r)
    return out


# speedup vs baseline: 1.8199x; 1.0073x over previous
import jax
import jax.numpy as jnp
from jax.experimental import pallas as pl
from jax.experimental.pallas import tpu as pltpu

_E, _CAP, _D = 64, 1024, 512
_H = 4 * _D
_RC = 256
_NC = _CAP // _RC


def _ffn_body(vl_ref, x_ref, w1_ref, b1_ref, w2_ref, b2_ref, o_ref):
    e = pl.program_id(0)
    valid = vl_ref[e]

    for k in range(_NC):
        base = k * _RC
        rows = slice(base, base + _RC)

        @pl.when(base < valid)
        def _compute(rows=rows, base=base):
            x = x_ref[0, rows, :]
            h = jnp.dot(x, w1_ref[0], preferred_element_type=jnp.float32)
            h = h + b1_ref[0]
            h = 0.5 * h * (1.0 + jax.lax.erf(h * 0.7071067811865476))
            y = jnp.dot(h, w2_ref[0], preferred_element_type=jnp.float32)
            y = y + b2_ref[0]
            ridx = base + jax.lax.broadcasted_iota(jnp.int32, (_RC, 1), 0)
            o_ref[0, rows, :] = jnp.where(ridx < valid, y, 0.0)

        @pl.when(base >= valid)
        def _zero(rows=rows):
            o_ref[0, rows, :] = jnp.zeros((_RC, _D), jnp.float32)


def kernel(packed_inputs, valid_load, w1, b1, w2, b2):
    vl = valid_load.astype(jnp.int32)
    b1r = b1.reshape(_E, 1, _H)
    b2r = b2.reshape(_E, 1, _D)

    def _emap(e, vl_ref):
        return (e, 0, 0)

    out = pl.pallas_call(
        _ffn_body,
        out_shape=jax.ShapeDtypeStruct((_E, _CAP, _D), jnp.float32),
        grid_spec=pltpu.PrefetchScalarGridSpec(
            num_scalar_prefetch=1,
            grid=(_E,),
            in_specs=[
                pl.BlockSpec((1, _CAP, _D), _emap),
                pl.BlockSpec((1, _D, _H), _emap),
                pl.BlockSpec((1, 1, _H), _emap),
                pl.BlockSpec((1, _H, _D), _emap),
                pl.BlockSpec((1, 1, _D), _emap),
            ],
            out_specs=pl.BlockSpec((1, _CAP, _D), _emap),
        ),
        compiler_params=pltpu.CompilerParams(
            dimension_semantics=("parallel",),
            vmem_limit_bytes=56 * 1024 * 1024,
        ),
        name="fused_group_ffn",
    )(vl, packed_inputs, w1, b1r, w2, b2r)
    return out


# speedup vs baseline: 1.8396x; 1.0108x over previous
import jax
import jax.numpy as jnp
from jax.experimental import pallas as pl
from jax.experimental.pallas import tpu as pltpu

_E, _CAP, _D = 64, 1024, 512
_H = 4 * _D
_RC = 256
_NC = _CAP // _RC


def _ffn_body(sp_ref, x0, x1, x2, x3, w1_ref, b1_ref, w2_ref, b2_ref, o_ref):
    e = pl.program_id(0)
    valid = sp_ref[0, e]
    xs = (x0, x1, x2, x3)

    for k in range(_NC):
        base = k * _RC
        rows = slice(base, base + _RC)

        @pl.when(base < valid)
        def _compute(rows=rows, base=base, x_ref=xs[k]):
            x = x_ref[0, 0]
            h = jnp.dot(x, w1_ref[0], preferred_element_type=jnp.float32)
            h = h + b1_ref[0]
            h = 0.5 * h * (1.0 + jax.lax.erf(h * 0.7071067811865476))
            y = jnp.dot(h, w2_ref[0], preferred_element_type=jnp.float32)
            y = y + b2_ref[0]
            ridx = base + jax.lax.broadcasted_iota(jnp.int32, (_RC, 1), 0)
            o_ref[0, rows, :] = jnp.where(ridx < valid, y, 0.0)

        @pl.when(base >= valid)
        def _zero(rows=rows):
            o_ref[0, rows, :] = jnp.zeros((_RC, _D), jnp.float32)


def kernel(packed_inputs, valid_load, w1, b1, w2, b2):
    vl = valid_load.astype(jnp.int32)

    eids = jnp.arange(_E, dtype=jnp.int32)
    thresh = jnp.array([k * _RC for k in range(_NC)], jnp.int32)
    chunk_valid = vl[None, :] > thresh[:, None]
    any_valid = (vl > 0)[None, :]
    live = jnp.concatenate([chunk_valid, any_valid], axis=0)
    maps = jax.lax.cummax(jnp.where(live, eids[None, :], 0), axis=1)
    sp = jnp.concatenate([vl[None, :], maps], axis=0)

    xr = packed_inputs.reshape(_E, _NC, _RC, _D)
    b1r = b1.reshape(_E, 1, _H)
    b2r = b2.reshape(_E, 1, _D)

    def _xmap(k):
        return lambda e, sp_ref: (sp_ref[1 + k, e], k, 0, 0)

    def _wmap(e, sp_ref):
        return (sp_ref[1 + _NC, e], 0, 0)

    out = pl.pallas_call(
        _ffn_body,
        out_shape=jax.ShapeDtypeStruct((_E, _CAP, _D), jnp.float32),
        grid_spec=pltpu.PrefetchScalarGridSpec(
            num_scalar_prefetch=1,
            grid=(_E,),
            in_specs=[
                pl.BlockSpec((1, 1, _RC, _D), _xmap(0)),
                pl.BlockSpec((1, 1, _RC, _D), _xmap(1)),
                pl.BlockSpec((1, 1, _RC, _D), _xmap(2)),
                pl.BlockSpec((1, 1, _RC, _D), _xmap(3)),
                pl.BlockSpec((1, _D, _H), _wmap),
                pl.BlockSpec((1, 1, _H), _wmap),
                pl.BlockSpec((1, _H, _D), _wmap),
                pl.BlockSpec((1, 1, _D), _wmap),
            ],
            out_specs=pl.BlockSpec((1, _CAP, _D), lambda e, sp_ref: (e, 0, 0)),
        ),
        compiler_params=pltpu.CompilerParams(
            dimension_semantics=("parallel",),
            vmem_limit_bytes=56 * 1024 * 1024,
        ),
        name="fused_group_ffn",
    )(sp, xr, xr, xr, xr, w1, b1r, w2, b2r)
    return out
